# baseline (speedup 1.0000x reference)
import jax
import jax.numpy as jnp
from jax import lax
from jax.experimental import pallas as pl
from jax.experimental.pallas import tpu as pltpu
from jax.experimental.pallas import tpu_sc as plsc

V = 1_000_000
F = 32
B = 4096
H = 50

NC = 2
NS = 16
NW = NC * NS

W = 512
NFULL = V // W
TAIL = V - NFULL * W
WIN_BASE = NFULL // NW
BB = B // NW

_MESH = plsc.VectorSubcoreMesh(core_axis_name="c", subcore_axis_name="s")


def _body(table_t, idx_t, out, scratch, in_v, tr_v, in_t, tr_t, idx1d,
          rows_v, fstage, dma_sem, gsem):
    cid = lax.axis_index("c")
    sid = lax.axis_index("s")
    wid = sid * NC + cid
    iota = lax.iota(jnp.int32, 16)

    def do_window(v0, src, dst, width):
        pltpu.sync_copy(table_t.at[:, pl.ds(v0, width)], src)

        def step(v, _):
            lo = plsc.load_gather(src, [iota, jnp.full((16,), v, jnp.int32)])
            hi = plsc.load_gather(src, [iota + 16, jnp.full((16,), v, jnp.int32)])
            dst[v, pl.ds(0, 16)] = lo
            dst[v, pl.ds(16, 16)] = hi
            return ()

        lax.fori_loop(0, width, step, ())
        pltpu.sync_copy(dst, scratch.at[pl.ds(v0, width)])

    nwin = WIN_BASE + jnp.where(wid < NFULL - WIN_BASE * NW, 1, 0)

    def win_step(i, _):
        do_window((wid + i * NW) * W, in_v, tr_v, W)
        return ()

    lax.fori_loop(0, nwin, win_step, ())

    @pl.when(wid == 1)
    def _():
        do_window(NFULL * W, in_t, tr_t, TAIL)

    plsc.subcore_barrier()

    @pl.when(sid == 0)
    def _():
        pl.semaphore_signal(gsem, 1, core_index=1 - cid)
        pl.semaphore_wait(gsem, 1)

    plsc.subcore_barrier()

    b0 = wid * BB
    for h in range(H):
        pltpu.sync_copy(idx_t.at[h, pl.ds(b0, BB)], idx1d.at[pl.ds(h * BB, BB)])

    def h_step(h, _):
        pltpu.async_copy(
            scratch.at[idx1d.at[pl.ds(h * BB, BB)]], rows_v, dma_sem
        ).wait()

        def f_step(f, _):
            for k in range(BB // 16):
                vals = plsc.load_gather(
                    rows_v, [iota + 16 * k, jnp.full((16,), f, jnp.int32)]
                )
                fstage[f, pl.ds(16 * k, 16)] = vals
            return ()

        lax.fori_loop(0, F, f_step, ())
        pltpu.sync_copy(fstage, out.at[h, :, pl.ds(b0, BB)])
        return ()

    lax.fori_loop(0, H, h_step, ())


_embed = pl.kernel(
    _body,
    out_type=jax.ShapeDtypeStruct((H, F, B), jnp.float32),
    mesh=_MESH,
    scratch_types=[
        pltpu.HBM((V, F), jnp.float32),
        pltpu.VMEM((F, W), jnp.float32),
        pltpu.VMEM((W, F), jnp.float32),
        pltpu.VMEM((F, TAIL), jnp.float32),
        pltpu.VMEM((TAIL, F), jnp.float32),
        pltpu.VMEM((H * BB,), jnp.int32),
        pltpu.VMEM((BB, F), jnp.float32),
        pltpu.VMEM((F, BB), jnp.float32),
        pltpu.SemaphoreType.DMA,
        pltpu.SemaphoreType.REGULAR,
    ],
    compiler_params=pltpu.CompilerParams(
        use_tc_tiling_on_sc=True, needs_layout_passes=False
    ),
)


def kernel(inputs, embedding):
    out = _embed(embedding.T, inputs.T)
    return out.transpose(2, 0, 1)


# speedup vs baseline: 1.4794x; 1.4794x over previous
import jax
import jax.numpy as jnp
from jax import lax
from jax.experimental import pallas as pl
from jax.experimental.pallas import tpu as pltpu
from jax.experimental.pallas import tpu_sc as plsc

NUM_EMB = 1_000_000
FEATURES = 32
BATCH = 4096
HIST = 50

NC = 2
NS = 16
NW = NC * NS

B_TOTAL = BATCH * HIST
B_PER_W = B_TOTAL // NW
CHUNK = 1_600
N_CHUNKS = B_PER_W // CHUNK

_MESH = plsc.VectorSubcoreMesh(core_axis_name="c", subcore_axis_name="s")


def _body(idx_hbm, table_hbm, out_hbm, i0, i1, r0, r1, is0, is1, gsem, ws0, ws1):
    wid = lax.axis_index("s") * NC + lax.axis_index("c")
    ibufs = (i0, i1)
    rbufs = (r0, r1)
    isems = (is0, is1)
    wsems = (ws0, ws1)
    icopies = [None, None]
    wcopies = [None, None]

    pltpu.sync_copy(idx_hbm.at[pl.ds(wid * B_PER_W, CHUNK)], ibufs[0])
    for j in range(N_CHUNKS):
        p = j % 2
        if icopies[p] is not None:
            icopies[p].wait()
        if j + 1 < N_CHUNKS:
            icopies[1 - p] = pltpu.async_copy(
                idx_hbm.at[pl.ds(wid * B_PER_W + (j + 1) * CHUNK, CHUNK)],
                ibufs[1 - p],
                isems[1 - p],
            )
        if wcopies[p] is not None:
            wcopies[p].wait()
        pltpu.async_copy(table_hbm.at[ibufs[p]], rbufs[p], gsem).wait()
        wcopies[p] = pltpu.async_copy(
            rbufs[p],
            out_hbm.at[pl.ds(wid * B_PER_W + j * CHUNK, CHUNK)],
            wsems[p],
        )
    for w in wcopies:
        if w is not None:
            w.wait()


_gather = pl.kernel(
    _body,
    out_type=jax.ShapeDtypeStruct((B_TOTAL, FEATURES), jnp.float32),
    mesh=_MESH,
    scratch_types=[
        pltpu.VMEM((CHUNK,), jnp.int32),
        pltpu.VMEM((CHUNK,), jnp.int32),
        pltpu.VMEM((CHUNK, FEATURES), jnp.float32),
        pltpu.VMEM((CHUNK, FEATURES), jnp.float32),
        pltpu.SemaphoreType.DMA,
        pltpu.SemaphoreType.DMA,
        pltpu.SemaphoreType.DMA,
        pltpu.SemaphoreType.DMA,
        pltpu.SemaphoreType.DMA,
    ],
    compiler_params=pltpu.CompilerParams(use_tc_tiling_on_sc=False),
)


def kernel(inputs, embedding):
    idx = inputs.reshape(-1).astype(jnp.int32)
    out = _gather(idx, embedding)
    return out.reshape(BATCH, HIST, FEATURES)
